# baseline (speedup 1.0000x reference)
import jax
import jax.numpy as jnp
from jax.experimental import pallas as pl
from jax.experimental.pallas import tpu as pltpu

B, N, D, H = 4, 2048, 128, 128
TILE = 1024
HALF = TILE // 2


def _fused_kernel(ga_ref, gb_ref, embf_ref, embr_ref, w1_ref, b1_ref,
                  w2_ref, b2_ref, ln_a_ref, ln_b_ref, fw1_ref, fb1_ref,
                  fw2_ref, fb2_ref, out_ref, x1_scratch, x2_scratch):
    p = pl.program_id(0)
    b = pl.program_id(1)
    t = pl.program_id(2)

    @pl.when(p == 0)
    def _phase0():
        @pl.when(t == 0)
        def _():
            x1_scratch[...] = (
                jnp.dot(embf_ref[0], w1_ref[...],
                        preferred_element_type=jnp.float32) + b1_ref[...]
            ).astype(jnp.bfloat16)

        for i, gref in enumerate((ga_ref, gb_ref)):
            h = jnp.dot(gref[0].astype(jnp.bfloat16), x1_scratch[...],
                        preferred_element_type=jnp.float32)
            h = jnp.maximum(h, 0.0)
            x2 = jnp.dot(h, w2_ref[...],
                         preferred_element_type=jnp.float32) + b2_ref[...]
            x2_scratch[b, pl.ds(t * TILE + i * HALF, HALF), :] = (
                x2.astype(jnp.bfloat16))

    @pl.when(p == 1)
    def _phase1():
        eps = 1e-6
        for i, gref in enumerate((ga_ref, gb_ref)):
            temp = jnp.dot(gref[0].astype(jnp.bfloat16), x2_scratch[b],
                           preferred_element_type=jnp.float32)
            mean = jnp.mean(temp, axis=-1, keepdims=True)
            cent = temp - mean
            var = jnp.sum(cent * cent, axis=-1, keepdims=True) / (D - 1)
            std = jnp.sqrt(var)
            normed = ln_a_ref[...] * cent / (std + eps) + ln_b_ref[...]
            num_fea = normed + embr_ref[0, pl.ds(i * HALF, HALF), :]
            ff = jnp.dot(num_fea, fw1_ref[...],
                         preferred_element_type=jnp.float32) + fb1_ref[...]
            ff = jnp.maximum(ff, 0.0)
            ff = jnp.dot(ff, fw2_ref[...],
                         preferred_element_type=jnp.float32) + fb2_ref[...]
            out_ref[0, pl.ds(i * HALF, HALF), :] = ff + num_fea


@jax.jit
def kernel(emb, graph, gcn_W1, gcn_b1, gcn_W2, gcn_b2, ln_a, ln_b,
           ff_W1, ff_b1, ff_W2, ff_b2):
    grid = (2, B, N // TILE)
    out = pl.pallas_call(
        _fused_kernel,
        grid=grid,
        in_specs=[
            pl.BlockSpec((1, HALF, N), lambda p, b, t: (b, 2 * t, 0)),
            pl.BlockSpec((1, HALF, N), lambda p, b, t: (b, 2 * t + 1, 0)),
            pl.BlockSpec((1, N, D), lambda p, b, t: (b * (1 - p), 0, 0)),
            pl.BlockSpec((1, TILE, D), lambda p, b, t: (b * p, t * p, 0)),
            pl.BlockSpec((D, H), lambda p, b, t: (0, 0)),
            pl.BlockSpec((H,), lambda p, b, t: (0,)),
            pl.BlockSpec((H, D), lambda p, b, t: (0, 0)),
            pl.BlockSpec((D,), lambda p, b, t: (0,)),
            pl.BlockSpec((D,), lambda p, b, t: (0,)),
            pl.BlockSpec((D,), lambda p, b, t: (0,)),
            pl.BlockSpec((D, H), lambda p, b, t: (0, 0)),
            pl.BlockSpec((H,), lambda p, b, t: (0,)),
            pl.BlockSpec((H, D), lambda p, b, t: (0, 0)),
            pl.BlockSpec((D,), lambda p, b, t: (0,)),
        ],
        out_specs=pl.BlockSpec((1, TILE, D), lambda p, b, t: (b * p, t * p, 0)),
        out_shape=jax.ShapeDtypeStruct((B, N, D), jnp.float32),
        scratch_shapes=[pltpu.VMEM((N, H), jnp.bfloat16),
                        pltpu.VMEM((B, N, D), jnp.bfloat16)],
    )(graph, graph, emb, emb, gcn_W1, gcn_b1, gcn_W2, gcn_b2, ln_a, ln_b,
      ff_W1, ff_b1, ff_W2, ff_b2)
    return out


# speedup vs baseline: 1.2011x; 1.2011x over previous
import jax
import jax.numpy as jnp
from jax.experimental import pallas as pl
from jax.experimental.pallas import tpu as pltpu

B, N, D, H = 4, 2048, 128, 128
TILE = 2048


def _fused_kernel(graph_ref, embf_ref, embr_ref, w1_ref, b1_ref, w2_ref,
                  b2_ref, ln_a_ref, ln_b_ref, fw1_ref, fb1_ref, fw2_ref,
                  fb2_ref, out_ref, x1_scratch, x2_scratch):
    p = pl.program_id(0)
    b = pl.program_id(1)
    t = pl.program_id(2)
    gb = graph_ref[0].astype(jnp.bfloat16)

    @pl.when(p == 0)
    def _phase0():
        @pl.when(t == 0)
        def _():
            x1_scratch[...] = (
                jnp.dot(embf_ref[0], w1_ref[...],
                        preferred_element_type=jnp.float32) + b1_ref[...]
            ).astype(jnp.bfloat16)

        h = jnp.dot(gb, x1_scratch[...], preferred_element_type=jnp.float32)
        h = jnp.maximum(h, 0.0)
        x2 = jnp.dot(h, w2_ref[...],
                     preferred_element_type=jnp.float32) + b2_ref[...]
        x2_scratch[b, pl.ds(t * TILE, TILE), :] = x2.astype(jnp.bfloat16)

    @pl.when(p == 1)
    def _phase1():
        eps = 1e-6
        temp = jnp.dot(gb, x2_scratch[b],
                       preferred_element_type=jnp.float32)
        mean = jnp.mean(temp, axis=-1, keepdims=True)
        cent = temp - mean
        var = jnp.sum(cent * cent, axis=-1, keepdims=True) / (D - 1)
        std = jnp.sqrt(var)
        normed = ln_a_ref[...] * cent / (std + eps) + ln_b_ref[...]
        num_fea = normed + embr_ref[0]
        ff = jnp.dot(num_fea, fw1_ref[...],
                     preferred_element_type=jnp.float32) + fb1_ref[...]
        ff = jnp.maximum(ff, 0.0)
        ff = jnp.dot(ff, fw2_ref[...],
                     preferred_element_type=jnp.float32) + fb2_ref[...]
        out_ref[0] = ff + num_fea


@jax.jit
def kernel(emb, graph, gcn_W1, gcn_b1, gcn_W2, gcn_b2, ln_a, ln_b,
           ff_W1, ff_b1, ff_W2, ff_b2):
    grid = (2, B, N // TILE)
    out = pl.pallas_call(
        _fused_kernel,
        grid=grid,
        in_specs=[
            pl.BlockSpec((1, TILE, N), lambda p, b, t: (b, t, 0)),
            pl.BlockSpec((1, N, D), lambda p, b, t: (b * (1 - p), 0, 0)),
            pl.BlockSpec((1, TILE, D), lambda p, b, t: (b * p, t * p, 0)),
            pl.BlockSpec((D, H), lambda p, b, t: (0, 0)),
            pl.BlockSpec((H,), lambda p, b, t: (0,)),
            pl.BlockSpec((H, D), lambda p, b, t: (0, 0)),
            pl.BlockSpec((D,), lambda p, b, t: (0,)),
            pl.BlockSpec((D,), lambda p, b, t: (0,)),
            pl.BlockSpec((D,), lambda p, b, t: (0,)),
            pl.BlockSpec((D, H), lambda p, b, t: (0, 0)),
            pl.BlockSpec((H,), lambda p, b, t: (0,)),
            pl.BlockSpec((H, D), lambda p, b, t: (0, 0)),
            pl.BlockSpec((D,), lambda p, b, t: (0,)),
        ],
        out_specs=pl.BlockSpec((1, TILE, D), lambda p, b, t: (b * p, t * p, 0)),
        out_shape=jax.ShapeDtypeStruct((B, N, D), jnp.float32),
        scratch_shapes=[pltpu.VMEM((N, H), jnp.bfloat16),
                        pltpu.VMEM((B, N, D), jnp.bfloat16)],
    )(graph, emb, emb, gcn_W1, gcn_b1, gcn_W2, gcn_b2, ln_a, ln_b,
      ff_W1, ff_b1, ff_W2, ff_b2)
    return out


# speedup vs baseline: 1.2882x; 1.0726x over previous
import jax
import jax.numpy as jnp
from jax.experimental import pallas as pl

B, N, D, H = 4, 2048, 128, 128


def _fused_kernel(graph_ref, emb_ref, w1_ref, b1_ref, w2_ref, b2_ref,
                  ln_a_ref, ln_b_ref, fw1_ref, fb1_ref, fw2_ref, fb2_ref,
                  out_ref):
    eps = 1e-6
    gb = graph_ref[0].astype(jnp.bfloat16)
    emb = emb_ref[0]
    x1 = (jnp.dot(emb, w1_ref[...],
                  preferred_element_type=jnp.float32) + b1_ref[...]
          ).astype(jnp.bfloat16)
    h = jnp.dot(gb, x1, preferred_element_type=jnp.float32)
    h = jnp.maximum(h, 0.0)
    x2 = (jnp.dot(h, w2_ref[...],
                  preferred_element_type=jnp.float32) + b2_ref[...]
          ).astype(jnp.bfloat16)
    temp = jnp.dot(gb, x2, preferred_element_type=jnp.float32)
    mean = jnp.mean(temp, axis=-1, keepdims=True)
    cent = temp - mean
    var = jnp.sum(cent * cent, axis=-1, keepdims=True) / (D - 1)
    std = jnp.sqrt(var)
    normed = ln_a_ref[...] * cent / (std + eps) + ln_b_ref[...]
    num_fea = normed + emb
    ff = jnp.dot(num_fea, fw1_ref[...],
                 preferred_element_type=jnp.float32) + fb1_ref[...]
    ff = jnp.maximum(ff, 0.0)
    ff = jnp.dot(ff, fw2_ref[...],
                 preferred_element_type=jnp.float32) + fb2_ref[...]
    out_ref[0] = ff + num_fea


@jax.jit
def kernel(emb, graph, gcn_W1, gcn_b1, gcn_W2, gcn_b2, ln_a, ln_b,
           ff_W1, ff_b1, ff_W2, ff_b2):
    out = pl.pallas_call(
        _fused_kernel,
        grid=(B,),
        in_specs=[
            pl.BlockSpec((1, N, N), lambda b: (b, 0, 0)),
            pl.BlockSpec((1, N, D), lambda b: (b, 0, 0)),
            pl.BlockSpec((D, H), lambda b: (0, 0)),
            pl.BlockSpec((H,), lambda b: (0,)),
            pl.BlockSpec((H, D), lambda b: (0, 0)),
            pl.BlockSpec((D,), lambda b: (0,)),
            pl.BlockSpec((D,), lambda b: (0,)),
            pl.BlockSpec((D,), lambda b: (0,)),
            pl.BlockSpec((D, H), lambda b: (0, 0)),
            pl.BlockSpec((H,), lambda b: (0,)),
            pl.BlockSpec((H, D), lambda b: (0, 0)),
            pl.BlockSpec((D,), lambda b: (0,)),
        ],
        out_specs=pl.BlockSpec((1, N, D), lambda b: (b, 0, 0)),
        out_shape=jax.ShapeDtypeStruct((B, N, D), jnp.float32),
    )(graph, emb, gcn_W1, gcn_b1, gcn_W2, gcn_b2, ln_a, ln_b,
      ff_W1, ff_b1, ff_W2, ff_b2)
    return out
